# baseline (speedup 1.0000x reference)
import functools

import jax
import jax.numpy as jnp
from jax import lax
from jax.experimental import pallas as pl
from jax.experimental.pallas import tpu as pltpu
from jax.experimental.pallas import tpu_sc as plsc

N = 10000
E = 320000
H = 128

NC = 2
NS = 16
NW = NC * NS

K = 64
CHUNKS = 158
EPT = CHUNKS * K
E_PAD = NW * EPT

KD = 128
CHD = 80
EPTD = CHD * KD
E_PADD = NW * EPTD

N_PAD = 10240
ROWS_PER_TILE = N_PAD // NS
ZCH = ROWS_PER_TILE // KD


@functools.lru_cache(maxsize=1)
def _mesh():
    return plsc.VectorSubcoreMesh(core_axis_name="c", subcore_axis_name="s")


def _sc_deg(dst_p, ones128, zeros128):
    @pl.kernel(
        out_type=jax.ShapeDtypeStruct((NC, N_PAD, H), jnp.float32),
        mesh=_mesh(),
        scratch_types=[
            pltpu.VMEM((CHD, KD), jnp.int32),
            pltpu.VMEM((KD, H), jnp.float32),
            pltpu.VMEM_SHARED((N_PAD, H), jnp.float32),
            pltpu.SemaphoreType.DMA,
        ],
    )
    def deg_kernel(dst_hbm, ones_hbm, zeros_hbm, out_hbm, dst_v, row_v, accum, sem):
        c = lax.axis_index("c")
        s = lax.axis_index("s")
        tile = c * NS + s
        pltpu.sync_copy(zeros_hbm, row_v)

        @pl.loop(0, ZCH)
        def _zero(j):
            pltpu.sync_copy(row_v, accum.at[pl.ds(s * ROWS_PER_TILE + j * KD, KD)])

        pltpu.sync_copy(dst_hbm.at[tile], dst_v)
        pltpu.sync_copy(ones_hbm, row_v)
        plsc.subcore_barrier()

        @pl.loop(0, CHD)
        def _body(i):
            pltpu.sync_copy(row_v, accum.at[dst_v.at[i]], add=True)

        plsc.subcore_barrier()

        @pl.loop(0, ZCH)
        def _out(j):
            r0 = s * ROWS_PER_TILE + j * KD
            pltpu.sync_copy(accum.at[pl.ds(r0, KD)], out_hbm.at[c, pl.ds(r0, KD)])

    return deg_kernel(dst_p, ones128, zeros128)


CH0 = 220
CH1 = 2 * CHUNKS - CH0
CH_MAX = max(CH0, CH1)
TCH = 2 * CHUNKS * NS


NBUF = 4


def _sc_agg(y, src_p, dst_p, zeros64):
    @pl.kernel(
        out_type=jax.ShapeDtypeStruct((NC, N_PAD, H), jnp.float32),
        mesh=_mesh(),
        scratch_types=(
            [pltpu.VMEM((CH_MAX * K,), jnp.int32)]
            + [pltpu.VMEM((K,), jnp.int32) for _ in range(NBUF)]
            + [pltpu.VMEM((K, H), jnp.float32) for _ in range(NBUF)]
            + [pltpu.VMEM_SHARED((N_PAD, H), jnp.float32)]
            + [pltpu.SemaphoreType.DMA for _ in range(2 * NBUF)]
        ),
    )
    def agg_kernel(y_hbm, src_hbm, dst_hbm, zeros_hbm, out_hbm, src_v, *rest):
        dsts = rest[0:NBUF]
        rows = rest[NBUF:2 * NBUF]
        accum = rest[2 * NBUF]
        sg = rest[2 * NBUF + 1: 3 * NBUF + 1]
        sd = rest[3 * NBUF + 1: 4 * NBUF + 1]
        c = lax.axis_index("c")
        s = lax.axis_index("s")
        pltpu.sync_copy(zeros_hbm, rows[0])

        @pl.loop(0, ROWS_PER_TILE // K)
        def _zero(j):
            pltpu.sync_copy(rows[0], accum.at[pl.ds(s * ROWS_PER_TILE + j * K, K)])

        def run(start, n):
            e0 = start * K

            def src_at(i):
                return src_v.at[pl.ds(i * K, K)]

            pltpu.sync_copy(src_hbm.at[pl.ds(e0, n * K)], src_v.at[pl.ds(0, n * K)])
            for b in range(NBUF):
                pltpu.async_copy(dst_hbm.at[pl.ds(e0 + b * K, K)], dsts[b], sd[b])
                pltpu.async_copy(y_hbm.at[src_at(b)], rows[b], sg[b])

            @pl.loop(0, n - NBUF, step=NBUF)
            def _body(i):
                for b in range(NBUF):
                    pltpu.make_async_copy(y_hbm.at[src_at(i + b)], rows[b], sg[b]).wait()
                    pltpu.make_async_copy(dst_hbm.at[pl.ds(e0, K)], dsts[b], sd[b]).wait()
                    pltpu.sync_copy(rows[b], accum.at[dsts[b]], add=True)
                    pltpu.async_copy(y_hbm.at[src_at(i + b + NBUF)], rows[b], sg[b])
                    pltpu.async_copy(
                        dst_hbm.at[pl.ds(e0 + (i + b + NBUF) * K, K)], dsts[b], sd[b])

            for b in range(NBUF):
                pltpu.make_async_copy(y_hbm.at[src_at(n - NBUF + b)], rows[b], sg[b]).wait()
                pltpu.make_async_copy(dst_hbm.at[pl.ds(e0, K)], dsts[b], sd[b]).wait()
                pltpu.sync_copy(rows[b], accum.at[dsts[b]], add=True)

        @pl.when(c == 0)
        def _c0():
            run(s * CH0, CH0)

        @pl.when(c == 1)
        def _c1():
            run(NS * CH0 + s * CH1, CH1)

        plsc.subcore_barrier()

        @pl.loop(0, ZCH)
        def _out(j):
            r0 = s * ROWS_PER_TILE + j * KD
            pltpu.sync_copy(accum.at[pl.ds(r0, KD)], out_hbm.at[c, pl.ds(r0, KD)])

    return agg_kernel(y, src_p, dst_p, zeros64)


def _tc_call(body, out_shape, *args):
    return pl.pallas_call(body, out_shape=out_shape)(*args)


def _enc_body(x_ref, encW_ref, encb_ref, W1_ref, t_ref):
    h = jnp.dot(x_ref[...], encW_ref[...], preferred_element_type=jnp.float32)
    h = h + encb_ref[...]
    t_ref[...] = jnp.dot(h, W1_ref[...], preferred_element_type=jnp.float32)


def _dis_body(degp_ref, t_ref, dis_ref, y_ref):
    d = degp_ref[0, :N, 0:1] + degp_ref[1, :N, 0:1] + 1.0
    dis = lax.rsqrt(d)
    dis_ref[...] = dis
    y_ref[...] = t_ref[...] * dis


def _comb_body(agg_ref, y_ref, dis_ref, b_ref, W_ref, yn_ref):
    a = agg_ref[0, :N, :] + agg_ref[1, :N, :] + y_ref[...]
    h = jnp.maximum(a * dis_ref[...] + b_ref[...], 0.0)
    yn_ref[...] = jnp.dot(h, W_ref[...], preferred_element_type=jnp.float32) * dis_ref[...]


def _dec_body(agg_ref, y_ref, dis_ref, b_ref, decW_ref, decb_ref, o_ref):
    a = agg_ref[0, :N, :] + agg_ref[1, :N, :] + y_ref[...]
    h = jnp.maximum(a * dis_ref[...] + b_ref[...], 0.0)
    o = jnp.dot(h, decW_ref[...], preferred_element_type=jnp.float32)
    o_ref[...] = o + decb_ref[...]


def kernel(x, edge_index, enc_W, enc_b, W1, b1, W2, b2, W3, b3, dec_W, dec_b):
    src = edge_index[0]
    dst = edge_index[1]
    src_p = jnp.concatenate([src, jnp.zeros((E_PAD - E,), src.dtype)])
    dst_p = jnp.concatenate([dst, jnp.full((E_PAD - E,), N, dst.dtype)])
    dst_p3 = jnp.concatenate(
        [dst, jnp.full((E_PADD - E,), N, dst.dtype)]).reshape(NW, CHD, KD)

    zeros64 = jnp.zeros((K, H), jnp.float32)
    zeros128 = jnp.zeros((KD, H), jnp.float32)
    ones128 = jnp.ones((KD, H), jnp.float32)

    degp = _sc_deg(dst_p3, ones128, zeros128)
    t1 = _tc_call(_enc_body, jax.ShapeDtypeStruct((N, H), jnp.float32),
                  x, enc_W, enc_b.reshape(1, H), W1)
    dis, y1 = _tc_call(_dis_body,
                       (jax.ShapeDtypeStruct((N, 1), jnp.float32),
                        jax.ShapeDtypeStruct((N, H), jnp.float32)),
                       degp, t1)
    a1 = _sc_agg(y1, src_p, dst_p, zeros64)
    y2 = _tc_call(_comb_body, jax.ShapeDtypeStruct((N, H), jnp.float32),
                  a1, y1, dis, b1.reshape(1, H), W2)
    a2 = _sc_agg(y2, src_p, dst_p, zeros64)
    y3 = _tc_call(_comb_body, jax.ShapeDtypeStruct((N, H), jnp.float32),
                  a2, y2, dis, b2.reshape(1, H), W3)
    a3 = _sc_agg(y3, src_p, dst_p, zeros64)
    out = _tc_call(_dec_body, jax.ShapeDtypeStruct((N, H), jnp.float32),
                   a3, y3, dis, b3.reshape(1, H), dec_W, dec_b.reshape(1, H))
    return out


# speedup vs baseline: 1.0127x; 1.0127x over previous
import functools

import jax
import jax.numpy as jnp
from jax import lax
from jax.experimental import pallas as pl
from jax.experimental.pallas import tpu as pltpu
from jax.experimental.pallas import tpu_sc as plsc

N = 10000
E = 320000
H = 128

NC = 2
NS = 16
NW = NC * NS

K = 32
CHUNKS = 316
EPT = CHUNKS * K
E_PAD = NW * EPT

KD = 128
CHD = 80
EPTD = CHD * KD
E_PADD = NW * EPTD

N_PAD = 10240
ROWS_PER_TILE = N_PAD // NS
ZCH = ROWS_PER_TILE // KD


@functools.lru_cache(maxsize=1)
def _mesh():
    return plsc.VectorSubcoreMesh(core_axis_name="c", subcore_axis_name="s")


def _sc_deg(dst_p, ones128, zeros128):
    @pl.kernel(
        out_type=jax.ShapeDtypeStruct((NC, N_PAD, H), jnp.float32),
        mesh=_mesh(),
        scratch_types=[
            pltpu.VMEM((CHD, KD), jnp.int32),
            pltpu.VMEM((KD, H), jnp.float32),
            pltpu.VMEM_SHARED((N_PAD, H), jnp.float32),
            pltpu.SemaphoreType.DMA,
        ],
    )
    def deg_kernel(dst_hbm, ones_hbm, zeros_hbm, out_hbm, dst_v, row_v, accum, sem):
        c = lax.axis_index("c")
        s = lax.axis_index("s")
        tile = c * NS + s
        pltpu.sync_copy(zeros_hbm, row_v)

        @pl.loop(0, ZCH)
        def _zero(j):
            pltpu.sync_copy(row_v, accum.at[pl.ds(s * ROWS_PER_TILE + j * KD, KD)])

        pltpu.sync_copy(dst_hbm.at[tile], dst_v)
        pltpu.sync_copy(ones_hbm, row_v)
        plsc.subcore_barrier()

        @pl.loop(0, CHD)
        def _body(i):
            pltpu.sync_copy(row_v, accum.at[dst_v.at[i]], add=True)

        plsc.subcore_barrier()

        @pl.loop(0, ZCH)
        def _out(j):
            r0 = s * ROWS_PER_TILE + j * KD
            pltpu.sync_copy(accum.at[pl.ds(r0, KD)], out_hbm.at[c, pl.ds(r0, KD)])

    return deg_kernel(dst_p, ones128, zeros128)


CH0 = 472
CH1 = 2 * CHUNKS - CH0
CH_MAX = max(CH0, CH1)
TCH = 2 * CHUNKS * NS


NBUF = 8


def _sc_agg(y, src_p, dst_p, zeros64):
    @pl.kernel(
        out_type=jax.ShapeDtypeStruct((NC, N_PAD, H), jnp.float32),
        mesh=_mesh(),
        scratch_types=(
            [pltpu.VMEM((CH_MAX * K,), jnp.int32)]
            + [pltpu.VMEM((K,), jnp.int32) for _ in range(NBUF)]
            + [pltpu.VMEM((K, H), jnp.float32) for _ in range(NBUF)]
            + [pltpu.VMEM_SHARED((N_PAD, H), jnp.float32)]
            + [pltpu.SemaphoreType.DMA for _ in range(2 * NBUF)]
        ),
    )
    def agg_kernel(y_hbm, src_hbm, dst_hbm, zeros_hbm, out_hbm, src_v, *rest):
        dsts = rest[0:NBUF]
        rows = rest[NBUF:2 * NBUF]
        accum = rest[2 * NBUF]
        sg = rest[2 * NBUF + 1: 3 * NBUF + 1]
        sd = rest[3 * NBUF + 1: 4 * NBUF + 1]
        c = lax.axis_index("c")
        s = lax.axis_index("s")
        pltpu.sync_copy(zeros_hbm, rows[0])

        @pl.loop(0, ROWS_PER_TILE // K)
        def _zero(j):
            pltpu.sync_copy(rows[0], accum.at[pl.ds(s * ROWS_PER_TILE + j * K, K)])

        def run(start, n):
            e0 = start * K

            def src_at(i):
                return src_v.at[pl.ds(i * K, K)]

            pltpu.sync_copy(src_hbm.at[pl.ds(e0, n * K)], src_v.at[pl.ds(0, n * K)])
            for b in range(NBUF):
                pltpu.async_copy(dst_hbm.at[pl.ds(e0 + b * K, K)], dsts[b], sd[b])
                pltpu.async_copy(y_hbm.at[src_at(b)], rows[b], sg[b])

            @pl.loop(0, n - NBUF, step=NBUF)
            def _body(i):
                for b in range(NBUF):
                    pltpu.make_async_copy(y_hbm.at[src_at(i + b)], rows[b], sg[b]).wait()
                    pltpu.make_async_copy(dst_hbm.at[pl.ds(e0, K)], dsts[b], sd[b]).wait()
                    pltpu.sync_copy(rows[b], accum.at[dsts[b]], add=True)
                    pltpu.async_copy(y_hbm.at[src_at(i + b + NBUF)], rows[b], sg[b])
                    pltpu.async_copy(
                        dst_hbm.at[pl.ds(e0 + (i + b + NBUF) * K, K)], dsts[b], sd[b])

            for b in range(NBUF):
                pltpu.make_async_copy(y_hbm.at[src_at(n - NBUF + b)], rows[b], sg[b]).wait()
                pltpu.make_async_copy(dst_hbm.at[pl.ds(e0, K)], dsts[b], sd[b]).wait()
                pltpu.sync_copy(rows[b], accum.at[dsts[b]], add=True)

        @pl.when(c == 0)
        def _c0():
            run(s * CH0, CH0)

        @pl.when(c == 1)
        def _c1():
            run(NS * CH0 + s * CH1, CH1)

        plsc.subcore_barrier()

        @pl.loop(0, ZCH)
        def _out(j):
            r0 = s * ROWS_PER_TILE + j * KD
            pltpu.sync_copy(accum.at[pl.ds(r0, KD)], out_hbm.at[c, pl.ds(r0, KD)])

    return agg_kernel(y, src_p, dst_p, zeros64)


def _tc_call(body, out_shape, *args):
    return pl.pallas_call(body, out_shape=out_shape)(*args)


def _enc_body(x_ref, encW_ref, encb_ref, W1_ref, t_ref):
    h = jnp.dot(x_ref[...], encW_ref[...], preferred_element_type=jnp.float32)
    h = h + encb_ref[...]
    t_ref[...] = jnp.dot(h, W1_ref[...], preferred_element_type=jnp.float32)


def _dis_body(degp_ref, t_ref, dis_ref, y_ref):
    d = degp_ref[0, :N, 0:1] + degp_ref[1, :N, 0:1] + 1.0
    dis = lax.rsqrt(d)
    dis_ref[...] = dis
    y_ref[...] = t_ref[...] * dis


def _comb_body(agg_ref, y_ref, dis_ref, b_ref, W_ref, yn_ref):
    a = agg_ref[0, :N, :] + agg_ref[1, :N, :] + y_ref[...]
    h = jnp.maximum(a * dis_ref[...] + b_ref[...], 0.0)
    yn_ref[...] = jnp.dot(h, W_ref[...], preferred_element_type=jnp.float32) * dis_ref[...]


def _dec_body(agg_ref, y_ref, dis_ref, b_ref, decW_ref, decb_ref, o_ref):
    a = agg_ref[0, :N, :] + agg_ref[1, :N, :] + y_ref[...]
    h = jnp.maximum(a * dis_ref[...] + b_ref[...], 0.0)
    o = jnp.dot(h, decW_ref[...], preferred_element_type=jnp.float32)
    o_ref[...] = o + decb_ref[...]


def kernel(x, edge_index, enc_W, enc_b, W1, b1, W2, b2, W3, b3, dec_W, dec_b):
    src = edge_index[0]
    dst = edge_index[1]
    src_p = jnp.concatenate([src, jnp.zeros((E_PAD - E,), src.dtype)])
    dst_p = jnp.concatenate([dst, jnp.full((E_PAD - E,), N, dst.dtype)])
    dst_p3 = jnp.concatenate(
        [dst, jnp.full((E_PADD - E,), N, dst.dtype)]).reshape(NW, CHD, KD)

    zeros64 = jnp.zeros((K, H), jnp.float32)
    zeros128 = jnp.zeros((KD, H), jnp.float32)
    ones128 = jnp.ones((KD, H), jnp.float32)

    degp = _sc_deg(dst_p3, ones128, zeros128)
    t1 = _tc_call(_enc_body, jax.ShapeDtypeStruct((N, H), jnp.float32),
                  x, enc_W, enc_b.reshape(1, H), W1)
    dis, y1 = _tc_call(_dis_body,
                       (jax.ShapeDtypeStruct((N, 1), jnp.float32),
                        jax.ShapeDtypeStruct((N, H), jnp.float32)),
                       degp, t1)
    a1 = _sc_agg(y1, src_p, dst_p, zeros64)
    y2 = _tc_call(_comb_body, jax.ShapeDtypeStruct((N, H), jnp.float32),
                  a1, y1, dis, b1.reshape(1, H), W2)
    a2 = _sc_agg(y2, src_p, dst_p, zeros64)
    y3 = _tc_call(_comb_body, jax.ShapeDtypeStruct((N, H), jnp.float32),
                  a2, y2, dis, b2.reshape(1, H), W3)
    a3 = _sc_agg(y3, src_p, dst_p, zeros64)
    out = _tc_call(_dec_body, jax.ShapeDtypeStruct((N, H), jnp.float32),
                   a3, y3, dis, b3.reshape(1, H), dec_W, dec_b.reshape(1, H))
    return out
